# baseline (speedup 1.0000x reference)
import functools

import jax
import jax.numpy as jnp
from jax import lax
from jax.experimental import pallas as pl
from jax.experimental.pallas import tpu as pltpu
from jax.experimental.pallas import tpu_sc as plsc

N_NODES = 10000
N_EDGES = 320000
D_IN = 128

NC = 2
NS = 16
NW = NC * NS
E_PER_W = N_EDGES // NW
CH = 80
N_STEPS = E_PER_W // CH
N_PAIRS = (N_STEPS + 1) // 2


def _sc_gather_mul(z_hbm, edge_hbm, x_hbm, idx_s, idx_d,
                   rs0, rd0, rs1, rd1, p0, p1,
                   gs0, gd0, gs1, gd1, ss0, ss1):
    wid = lax.axis_index("s") * NC + lax.axis_index("c")
    base = wid * E_PER_W

    rows = ((rs0, rd0, p0, gs0, gd0, ss0), (rs1, rd1, p1, gs1, gd1, ss1))

    pltpu.sync_copy(edge_hbm.at[pl.ds(base, E_PER_W)], idx_s)
    pltpu.sync_copy(edge_hbm.at[pl.ds(N_EDGES + base, E_PER_W)], idx_d)

    def issue_gather(j, rs, rd, sg_s, sg_d):
        i_s = idx_s.at[pl.ds(j * CH, CH)]
        i_d = idx_d.at[pl.ds(j * CH, CH)]
        pltpu.async_copy(z_hbm.at[i_s], rs, sg_s)
        pltpu.async_copy(z_hbm.at[i_d], rd, sg_d)

    def wait_gather(j, rs, rd, sg_s, sg_d):
        i_s = idx_s.at[pl.ds(j * CH, CH)]
        i_d = idx_d.at[pl.ds(j * CH, CH)]
        pltpu.make_async_copy(z_hbm.at[i_s], rs, sg_s).wait()
        pltpu.make_async_copy(z_hbm.at[i_d], rd, sg_d).wait()

    def issue_scatter(j, p, sem):
        pltpu.async_copy(p, x_hbm.at[pl.ds(base + j * CH, CH)], sem)

    def wait_scatter(j, p, sem):
        pltpu.make_async_copy(p, x_hbm.at[pl.ds(base + j * CH, CH)], sem).wait()

    def mul(rs, rd, p):
        def mul_row(r, _):
            for k in range(D_IN // 16):
                sl = pl.ds(k * 16, 16)
                p[r, sl] = rs[r, sl] * rd[r, sl]
            return 0

        lax.fori_loop(0, CH, mul_row, 0, unroll=2)

    issue_gather(0, rs0, rd0, gs0, gd0)
    issue_gather(1, rs1, rd1, gs1, gd1)

    def pair(i, _):
        for b in range(2):
            rs, rd, p, sg_s, sg_d, ssem = rows[b]
            j = 2 * i + b

            def do_step():
                wait_gather(j, rs, rd, sg_s, sg_d)

                @pl.when(i > 0)
                def _():
                    wait_scatter(j - 2, p, ssem)

                mul(rs, rd, p)

                @pl.when(j + 2 < N_STEPS)
                def _():
                    issue_gather(j + 2, rs, rd, sg_s, sg_d)

                issue_scatter(j, p, ssem)

            if b == 0:
                do_step()
            else:
                pl.when(j < N_STEPS)(do_step)
        return 0

    lax.fori_loop(0, N_PAIRS, pair, 0)

    wait_scatter(N_STEPS - 1, p0, ss0)
    wait_scatter(N_STEPS - 2, p1, ss1)


def _gather_product(z, edge):
    mesh = plsc.VectorSubcoreMesh(core_axis_name="c", subcore_axis_name="s")
    f = functools.partial(
        pl.kernel,
        mesh=mesh,
        out_type=jax.ShapeDtypeStruct((N_EDGES, D_IN), jnp.float32),
        scratch_types=[
            pltpu.VMEM((E_PER_W,), jnp.int32),
            pltpu.VMEM((E_PER_W,), jnp.int32),
            pltpu.VMEM((CH, D_IN), jnp.float32),
            pltpu.VMEM((CH, D_IN), jnp.float32),
            pltpu.VMEM((CH, D_IN), jnp.float32),
            pltpu.VMEM((CH, D_IN), jnp.float32),
            pltpu.VMEM((CH, D_IN), jnp.float32),
            pltpu.VMEM((CH, D_IN), jnp.float32),
            pltpu.SemaphoreType.DMA,
            pltpu.SemaphoreType.DMA,
            pltpu.SemaphoreType.DMA,
            pltpu.SemaphoreType.DMA,
            pltpu.SemaphoreType.DMA,
            pltpu.SemaphoreType.DMA,
        ],
    )(_sc_gather_mul)
    return f(z, edge.reshape(-1))


BE = 2560


def _mlp_body(x_ref, w1_ref, b1_ref, w2_ref, b2_ref, o_ref):
    h = jnp.dot(x_ref[...], w1_ref[...], preferred_element_type=jnp.float32)
    h = jnp.maximum(h + b1_ref[...], 0.0)
    o = jnp.sum(h * w2_ref[...], axis=1, keepdims=True) + b2_ref[...]
    o_ref[...] = jax.nn.sigmoid(o)


def _mlp(x, W1, b1, W2, b2):
    grid = (N_EDGES // BE,)
    return pl.pallas_call(
        _mlp_body,
        grid=grid,
        in_specs=[
            pl.BlockSpec((BE, D_IN), lambda i: (i, 0)),
            pl.BlockSpec((D_IN, 64), lambda i: (0, 0)),
            pl.BlockSpec((1, 64), lambda i: (0, 0)),
            pl.BlockSpec((1, 64), lambda i: (0, 0)),
            pl.BlockSpec((1, 1), lambda i: (0, 0)),
        ],
        out_specs=pl.BlockSpec((BE, 1), lambda i: (i, 0)),
        out_shape=jax.ShapeDtypeStruct((N_EDGES, 1), jnp.float32),
    )(x, W1, b1.reshape(1, 64), W2.reshape(1, 64), b2.reshape(1, 1))


def kernel(z, edge, W1, b1, W2, b2):
    x = _gather_product(z, edge)
    return _mlp(x, W1, b1, W2, b2)


# speedup vs baseline: 1.0904x; 1.0904x over previous
import functools

import jax
import jax.numpy as jnp
from jax import lax
from jax.experimental import pallas as pl
from jax.experimental.pallas import tpu as pltpu
from jax.experimental.pallas import tpu_sc as plsc

N_NODES = 10000
N_EDGES = 320000
D_IN = 128

NC = 2
NS = 16
NW = NC * NS
E_PER_W = N_EDGES // NW
CH = 80
N_STEPS = E_PER_W // CH
N_PAIRS = (N_STEPS + 1) // 2


def _sc_gather_mul(z_hbm, edge_hbm, x_hbm, idx_s, idx_d,
                   rs0, rd0, rs1, rd1, p0, p1,
                   gs0, gd0, gs1, gd1, ss0, ss1):
    wid = lax.axis_index("s") * NC + lax.axis_index("c")
    base = wid * E_PER_W

    rows = ((rs0, rd0, p0, gs0, gd0, ss0), (rs1, rd1, p1, gs1, gd1, ss1))

    pltpu.sync_copy(edge_hbm.at[pl.ds(base, E_PER_W)], idx_s)
    pltpu.sync_copy(edge_hbm.at[pl.ds(N_EDGES + base, E_PER_W)], idx_d)

    def issue_gather(j, rs, rd, sg_s, sg_d):
        i_s = idx_s.at[pl.ds(j * CH, CH)]
        i_d = idx_d.at[pl.ds(j * CH, CH)]
        pltpu.async_copy(z_hbm.at[i_s], rs, sg_s)
        pltpu.async_copy(z_hbm.at[i_d], rd, sg_d)

    def wait_gather(j, rs, rd, sg_s, sg_d):
        i_s = idx_s.at[pl.ds(j * CH, CH)]
        i_d = idx_d.at[pl.ds(j * CH, CH)]
        pltpu.make_async_copy(z_hbm.at[i_s], rs, sg_s).wait()
        pltpu.make_async_copy(z_hbm.at[i_d], rd, sg_d).wait()

    def issue_scatter(j, p, sem):
        pltpu.async_copy(p, x_hbm.at[pl.ds(base + j * CH, CH)], sem)

    def wait_scatter(j, p, sem):
        pltpu.make_async_copy(p, x_hbm.at[pl.ds(base + j * CH, CH)], sem).wait()

    def mul(rs, rd, p):
        def mul_row(r, _):
            for k in range(D_IN // 16):
                sl = pl.ds(k * 16, 16)
                p[r, sl] = rs[r, sl] * rd[r, sl]
            return 0

        lax.fori_loop(0, CH, mul_row, 0, unroll=2)

    issue_gather(0, rs0, rd0, gs0, gd0)
    issue_gather(1, rs1, rd1, gs1, gd1)

    def pair(i, _):
        for b in range(2):
            rs, rd, p, sg_s, sg_d, ssem = rows[b]
            j = 2 * i + b

            def do_step():
                wait_gather(j, rs, rd, sg_s, sg_d)

                @pl.when(i > 0)
                def _():
                    wait_scatter(j - 2, p, ssem)

                mul(rs, rd, p)

                @pl.when(j + 2 < N_STEPS)
                def _():
                    issue_gather(j + 2, rs, rd, sg_s, sg_d)

                issue_scatter(j, p, ssem)

            if b == 0:
                do_step()
            else:
                pl.when(j < N_STEPS)(do_step)
        return 0

    lax.fori_loop(0, N_PAIRS, pair, 0)

    wait_scatter(N_STEPS - 1, p0, ss0)
    wait_scatter(N_STEPS - 2, p1, ss1)


def _gather_product(z, edge):
    mesh = plsc.VectorSubcoreMesh(core_axis_name="c", subcore_axis_name="s")
    f = functools.partial(
        pl.kernel,
        mesh=mesh,
        out_type=jax.ShapeDtypeStruct((N_EDGES, D_IN), jnp.float32),
        scratch_types=[
            pltpu.VMEM((E_PER_W,), jnp.int32),
            pltpu.VMEM((E_PER_W,), jnp.int32),
            pltpu.VMEM((CH, D_IN), jnp.float32),
            pltpu.VMEM((CH, D_IN), jnp.float32),
            pltpu.VMEM((CH, D_IN), jnp.float32),
            pltpu.VMEM((CH, D_IN), jnp.float32),
            pltpu.VMEM((CH, D_IN), jnp.float32),
            pltpu.VMEM((CH, D_IN), jnp.float32),
            pltpu.SemaphoreType.DMA,
            pltpu.SemaphoreType.DMA,
            pltpu.SemaphoreType.DMA,
            pltpu.SemaphoreType.DMA,
            pltpu.SemaphoreType.DMA,
            pltpu.SemaphoreType.DMA,
        ],
    )(_sc_gather_mul)
    return f(z, edge.reshape(-1))


BE = 8000


def _mlp_body(x_ref, w1_ref, b1_ref, w2_ref, b2_ref, o_ref):
    h = jnp.dot(x_ref[...], w1_ref[...], preferred_element_type=jnp.float32)
    h = jnp.maximum(h + b1_ref[...], 0.0)
    o = jnp.sum(h * w2_ref[...], axis=1, keepdims=True) + b2_ref[...]
    o_ref[...] = jax.nn.sigmoid(o)


def _mlp(x, W1, b1, W2, b2):
    grid = (N_EDGES // BE,)
    return pl.pallas_call(
        _mlp_body,
        grid=grid,
        in_specs=[
            pl.BlockSpec((BE, D_IN), lambda i: (i, 0)),
            pl.BlockSpec((D_IN, 64), lambda i: (0, 0)),
            pl.BlockSpec((1, 64), lambda i: (0, 0)),
            pl.BlockSpec((1, 64), lambda i: (0, 0)),
            pl.BlockSpec((1, 1), lambda i: (0, 0)),
        ],
        out_specs=pl.BlockSpec((BE, 1), lambda i: (i, 0)),
        out_shape=jax.ShapeDtypeStruct((N_EDGES, 1), jnp.float32),
    )(x, W1, b1.reshape(1, 64), W2.reshape(1, 64), b2.reshape(1, 1))


def kernel(z, edge, W1, b1, W2, b2):
    x = _gather_product(z, edge)
    return _mlp(x, W1, b1, W2, b2)


# speedup vs baseline: 1.0962x; 1.0053x over previous
import functools

import jax
import jax.numpy as jnp
from jax import lax
from jax.experimental import pallas as pl
from jax.experimental.pallas import tpu as pltpu
from jax.experimental.pallas import tpu_sc as plsc

N_NODES = 10000
N_EDGES = 320000
D_IN = 128

NC = 2
NS = 16
NW = NC * NS
E_PER_W = N_EDGES // NW
CH = 80
N_STEPS = E_PER_W // CH
N_PAIRS = (N_STEPS + 1) // 2


def _sc_gather_mul(z_hbm, edge_hbm, x_hbm, idx_s, idx_d,
                   rs0, rd0, rs1, rd1, p0, p1,
                   gs0, gd0, gs1, gd1, ss0, ss1):
    wid = lax.axis_index("s") * NC + lax.axis_index("c")
    base = wid * E_PER_W

    rows = ((rs0, rd0, p0, gs0, gd0, ss0), (rs1, rd1, p1, gs1, gd1, ss1))

    pltpu.sync_copy(edge_hbm.at[pl.ds(base, E_PER_W)], idx_s)
    pltpu.sync_copy(edge_hbm.at[pl.ds(N_EDGES + base, E_PER_W)], idx_d)

    def issue_gather(j, rs, rd, sg_s, sg_d):
        i_s = idx_s.at[pl.ds(j * CH, CH)]
        i_d = idx_d.at[pl.ds(j * CH, CH)]
        pltpu.async_copy(z_hbm.at[i_s], rs, sg_s)
        pltpu.async_copy(z_hbm.at[i_d], rd, sg_d)

    def wait_gather(j, rs, rd, sg_s, sg_d):
        i_s = idx_s.at[pl.ds(j * CH, CH)]
        i_d = idx_d.at[pl.ds(j * CH, CH)]
        pltpu.make_async_copy(z_hbm.at[i_s], rs, sg_s).wait()
        pltpu.make_async_copy(z_hbm.at[i_d], rd, sg_d).wait()

    def issue_scatter(j, p, sem):
        pltpu.async_copy(p, x_hbm.at[pl.ds(base + j * CH, CH)], sem)

    def wait_scatter(j, p, sem):
        pltpu.make_async_copy(p, x_hbm.at[pl.ds(base + j * CH, CH)], sem).wait()

    def mul(rs, rd, p):
        def mul_row(r, _):
            for k in range(D_IN // 16):
                sl = pl.ds(k * 16, 16)
                p[r, sl] = rs[r, sl] * rd[r, sl]
            return 0

        lax.fori_loop(0, CH, mul_row, 0, unroll=2)

    issue_gather(0, rs0, rd0, gs0, gd0)
    issue_gather(1, rs1, rd1, gs1, gd1)

    def pair(i, _):
        for b in range(2):
            rs, rd, p, sg_s, sg_d, ssem = rows[b]
            j = 2 * i + b

            def do_step():
                wait_gather(j, rs, rd, sg_s, sg_d)

                @pl.when(i > 0)
                def _():
                    wait_scatter(j - 2, p, ssem)

                mul(rs, rd, p)

                @pl.when(j + 2 < N_STEPS)
                def _():
                    issue_gather(j + 2, rs, rd, sg_s, sg_d)

                issue_scatter(j, p, ssem)

            if b == 0:
                do_step()
            else:
                pl.when(j < N_STEPS)(do_step)
        return 0

    lax.fori_loop(0, N_PAIRS, pair, 0)

    wait_scatter(N_STEPS - 1, p0, ss0)
    wait_scatter(N_STEPS - 2, p1, ss1)


def _gather_product(z, edge):
    mesh = plsc.VectorSubcoreMesh(core_axis_name="c", subcore_axis_name="s")
    f = functools.partial(
        pl.kernel,
        mesh=mesh,
        out_type=jax.ShapeDtypeStruct((N_EDGES, D_IN), jnp.float32),
        scratch_types=[
            pltpu.VMEM((E_PER_W,), jnp.int32),
            pltpu.VMEM((E_PER_W,), jnp.int32),
            pltpu.VMEM((CH, D_IN), jnp.float32),
            pltpu.VMEM((CH, D_IN), jnp.float32),
            pltpu.VMEM((CH, D_IN), jnp.float32),
            pltpu.VMEM((CH, D_IN), jnp.float32),
            pltpu.VMEM((CH, D_IN), jnp.float32),
            pltpu.VMEM((CH, D_IN), jnp.float32),
            pltpu.SemaphoreType.DMA,
            pltpu.SemaphoreType.DMA,
            pltpu.SemaphoreType.DMA,
            pltpu.SemaphoreType.DMA,
            pltpu.SemaphoreType.DMA,
            pltpu.SemaphoreType.DMA,
        ],
    )(_sc_gather_mul)
    return f(z, edge.reshape(-1))


BE = 16000


def _mlp_body(x_ref, w1_ref, b1_ref, w2_ref, b2_ref, o_ref):
    h = jnp.dot(x_ref[...], w1_ref[...], preferred_element_type=jnp.float32)
    h = jnp.maximum(h + b1_ref[...], 0.0)
    o = jnp.sum(h * w2_ref[...], axis=1, keepdims=True) + b2_ref[...]
    o_ref[...] = jax.nn.sigmoid(o)


def _mlp(x, W1, b1, W2, b2):
    grid = (N_EDGES // BE,)
    return pl.pallas_call(
        _mlp_body,
        grid=grid,
        in_specs=[
            pl.BlockSpec((BE, D_IN), lambda i: (i, 0)),
            pl.BlockSpec((D_IN, 64), lambda i: (0, 0)),
            pl.BlockSpec((1, 64), lambda i: (0, 0)),
            pl.BlockSpec((1, 64), lambda i: (0, 0)),
            pl.BlockSpec((1, 1), lambda i: (0, 0)),
        ],
        out_specs=pl.BlockSpec((BE, 1), lambda i: (i, 0)),
        out_shape=jax.ShapeDtypeStruct((N_EDGES, 1), jnp.float32),
    )(x, W1, b1.reshape(1, 64), W2.reshape(1, 64), b2.reshape(1, 1))


def kernel(z, edge, W1, b1, W2, b2):
    x = _gather_product(z, edge)
    return _mlp(x, W1, b1, W2, b2)
